# baseline (speedup 1.0000x reference)
import functools

import jax
import jax.numpy as jnp
from jax import lax
from jax.experimental import pallas as pl
from jax.experimental.pallas import tpu as pltpu
from jax.experimental.pallas import tpu_sc as plsc

_R, _C = 16384, 4096
_N = _R * _C
_LANES = 128
_ROWS2D = _N // _LANES
_NW = 32
_WROWS = _ROWS2D // _NW
_CHUNK_ROWS = 128
_NCHUNK = _WROWS // _CHUNK_ROWS
_COLP = _C // _LANES
_SHIFT = 12


def _sc_gather(x1d, idx2):
    mesh = plsc.VectorSubcoreMesh(core_axis_name="c", subcore_axis_name="s")

    @functools.partial(
        pl.kernel,
        mesh=mesh,
        out_type=jax.ShapeDtypeStruct((_ROWS2D, _LANES), jnp.float32),
        scratch_types=[
            pltpu.VMEM((_COLP, _LANES), jnp.int32),
            pltpu.VMEM((_CHUNK_ROWS, _LANES), jnp.int32),
            pltpu.VMEM((_CHUNK_ROWS, _LANES), jnp.int32),
            pltpu.VMEM((_CHUNK_ROWS, _LANES), jnp.float32),
            pltpu.SemaphoreType.DMA,
        ],
    )
    def k(x_hbm, idx_hbm, out_hbm, col_v, idx_v, fidx_v, data_v, sem):
        wid = lax.axis_index("s") * 2 + lax.axis_index("c")
        base_row = wid * _WROWS

        lane = lax.iota(jnp.int32, 16)
        for r in range(_COLP):
            for s in range(_LANES // 16):
                col_v[r, pl.ds(s * 16, 16)] = (r * _LANES + s * 16) % _C + lane

        def chunk_body(c, carry):
            row0 = base_row + c * _CHUNK_ROWS
            pltpu.sync_copy(idx_hbm.at[pl.ds(row0, _CHUNK_ROWS)], idx_v)

            def frow(r, carry2):
                rp = lax.rem(r, _COLP)
                for s in range(_LANES // 16):
                    sl = pl.ds(s * 16, 16)
                    fidx_v[r, sl] = (idx_v[r, sl] << _SHIFT) | col_v[rp, sl]
                return carry2

            lax.fori_loop(0, _CHUNK_ROWS, frow, 0, unroll=4)

            def gstart(g, carry2):
                pltpu.async_copy(x_hbm.at[fidx_v.at[g]], data_v.at[g], sem)
                return carry2

            lax.fori_loop(0, _CHUNK_ROWS, gstart, 0)
            pltpu.make_async_copy(
                out_hbm.at[pl.ds(0, _CHUNK_ROWS)], data_v, sem
            ).wait()
            pltpu.sync_copy(data_v, out_hbm.at[pl.ds(row0, _CHUNK_ROWS)])
            return carry

        lax.fori_loop(0, _NCHUNK, chunk_body, 0)

    return k(x1d, idx2)


def kernel(x, index):
    x1d = x.reshape(_N)
    idx2 = index.reshape(_ROWS2D, _LANES)
    out2 = _sc_gather(x1d, idx2)
    return out2.reshape(_R, _C)


# speedup vs baseline: 1.2557x; 1.2557x over previous
import functools

import jax
import jax.numpy as jnp
from jax import lax
from jax.experimental import pallas as pl
from jax.experimental.pallas import tpu as pltpu
from jax.experimental.pallas import tpu_sc as plsc

_R, _C = 16384, 4096
_N = _R * _C
_NW = 32
_PW = _N // _NW
_CHUNK = 16384
_NCHUNK = _PW // _CHUNK
_SHIFT = 12


def _sc_gather(x1d, idx1):
    mesh = plsc.VectorSubcoreMesh(core_axis_name="c", subcore_axis_name="s")

    @functools.partial(
        pl.kernel,
        mesh=mesh,
        out_type=jax.ShapeDtypeStruct((_N,), jnp.float32),
        scratch_types=[
            pltpu.VMEM((_C,), jnp.int32),
            pltpu.VMEM((_CHUNK,), jnp.int32),
            pltpu.VMEM((_CHUNK,), jnp.int32),
            pltpu.VMEM((_CHUNK,), jnp.int32),
            pltpu.VMEM((_CHUNK,), jnp.int32),
            pltpu.VMEM((_CHUNK,), jnp.float32),
            pltpu.VMEM((_CHUNK,), jnp.float32),
            pltpu.SemaphoreType.DMA,
            pltpu.SemaphoreType.DMA,
            pltpu.SemaphoreType.DMA,
        ],
    )
    def k(x_hbm, idx_hbm, out_hbm, col_v, idx_a, idx_b, fidx_a, fidx_b,
          data_a, data_b, sem_in, sem_g, sem_out):
        wid = lax.axis_index("s") * 2 + lax.axis_index("c")
        base = wid * _PW

        lane = lax.iota(jnp.int32, 16)

        def crow(r, carry):
            col_v[pl.ds(r * 16, 16)] = r * 16 + lane
            return carry

        lax.fori_loop(0, _C // 16, crow, 0, unroll=8)

        def idx_start(c, idx_v):
            pltpu.make_async_copy(
                idx_hbm.at[pl.ds(base + c * _CHUNK, _CHUNK)],
                idx_v, sem_in).start()

        def idx_wait(c, idx_v):
            pltpu.make_async_copy(
                idx_hbm.at[pl.ds(base + c * _CHUNK, _CHUNK)],
                idx_v, sem_in).wait()

        def fidx_compute(idx_v, fidx_v):
            def frow(r, carry):
                sl = pl.ds(r * 16, 16)
                cl = pl.ds(lax.rem(r * 16, _C), 16)
                fidx_v[sl] = (idx_v[sl] << _SHIFT) | col_v[cl]
                return carry
            lax.fori_loop(0, _CHUNK // 16, frow, 0, unroll=8)

        def gather_start(fidx_v, data_v):
            pltpu.make_async_copy(
                x_hbm.at[fidx_v], data_v, sem_g).start()

        def gather_wait(fidx_v, data_v):
            pltpu.make_async_copy(
                x_hbm.at[fidx_v], data_v, sem_g).wait()

        def out_start(c, data_v):
            pltpu.make_async_copy(
                data_v, out_hbm.at[pl.ds(base + c * _CHUNK, _CHUNK)],
                sem_out).start()

        def out_wait(c, data_v):
            pltpu.make_async_copy(
                data_v, out_hbm.at[pl.ds(base + c * _CHUNK, _CHUNK)],
                sem_out).wait()

        idx_start(0, idx_a)
        idx_start(1, idx_b)
        idx_wait(0, idx_a)
        fidx_compute(idx_a, fidx_a)
        gather_start(fidx_a, data_a)

        def half(c, cur, nxt):
            idx_c, fidx_c, data_c = cur
            idx_n, fidx_n, data_n = nxt

            @pl.when(c + 1 < _NCHUNK)
            def _stage_next():
                idx_wait(c + 1, idx_n)
                fidx_compute(idx_n, fidx_n)

            gather_wait(fidx_c, data_c)

            @pl.when(c > 0)
            def _drain_prev_out():
                out_wait(c - 1, data_n)

            @pl.when(c + 1 < _NCHUNK)
            def _fire_next():
                gather_start(fidx_n, data_n)

            out_start(c, data_c)

            @pl.when(c + 2 < _NCHUNK)
            def _prefetch():
                idx_start(c + 2, idx_c)

        bufs_a = (idx_a, fidx_a, data_a)
        bufs_b = (idx_b, fidx_b, data_b)

        def pair_body(cp, carry):
            half(2 * cp, bufs_a, bufs_b)
            half(2 * cp + 1, bufs_b, bufs_a)
            return carry

        lax.fori_loop(0, _NCHUNK // 2, pair_body, 0)
        out_wait(_NCHUNK - 1, data_b)

    return k(x1d, idx1)


def kernel(x, index):
    x1d = x.reshape(_N)
    idx1 = index.reshape(_N)
    out1 = _sc_gather(x1d, idx1)
    return out1.reshape(_R, _C)


# speedup vs baseline: 1.4636x; 1.1655x over previous
import functools

import jax
import jax.numpy as jnp
from jax import lax
from jax.experimental import pallas as pl
from jax.experimental.pallas import tpu as pltpu
from jax.experimental.pallas import tpu_sc as plsc

_R, _C = 16384, 4096
_N = _R * _C
_NW = 32
_WROWS = _R // _NW
_CR = 4
_CHUNK = _CR * _C
_NCHUNK = _WROWS // _CR
_SHIFT = 12


def _sc_gather(x1d, idx2):
    mesh = plsc.VectorSubcoreMesh(core_axis_name="c", subcore_axis_name="s")

    @functools.partial(
        pl.kernel,
        mesh=mesh,
        out_type=jax.ShapeDtypeStruct((_R, _C), jnp.float32),
        scratch_types=[
            pltpu.VMEM((_CHUNK,), jnp.int32),
            pltpu.VMEM((_CHUNK,), jnp.int32),
            pltpu.VMEM((_CHUNK,), jnp.int32),
            pltpu.VMEM((_CHUNK,), jnp.int32),
            pltpu.VMEM((_CHUNK,), jnp.float32),
            pltpu.VMEM((_CHUNK,), jnp.float32),
            pltpu.SemaphoreType.DMA,
            pltpu.SemaphoreType.DMA,
            pltpu.SemaphoreType.DMA,
        ],
    )
    def k(x_hbm, idx_hbm, out_hbm, idx_a, idx_b, fidx_a, fidx_b,
          data_a, data_b, sem_in, sem_g, sem_out):
        wid = lax.axis_index("s") * 2 + lax.axis_index("c")
        base = wid * _WROWS
        lane = lax.iota(jnp.int32, 16)

        def idx_start(c, idx_v):
            for r in range(_CR):
                pltpu.make_async_copy(
                    idx_hbm.at[base + c * _CR + r],
                    idx_v.at[pl.ds(r * _C, _C)], sem_in).start()

        def idx_wait(c, idx_v):
            for r in range(_CR):
                pltpu.make_async_copy(
                    idx_hbm.at[base + c * _CR + r],
                    idx_v.at[pl.ds(r * _C, _C)], sem_in).wait()

        def fidx_compute(idx_v, fidx_v):
            def frow(r, carry):
                sl = pl.ds(r * 16, 16)
                col = (lax.rem(r, _C // 16) << 4) + lane
                fidx_v[sl] = (idx_v[sl] << _SHIFT) | col
                return carry
            lax.fori_loop(0, _CHUNK // 16, frow, 0, unroll=8)

        def gather_start(fidx_v, data_v):
            pltpu.make_async_copy(
                x_hbm.at[fidx_v], data_v, sem_g).start()

        def gather_wait(fidx_v, data_v):
            pltpu.make_async_copy(
                x_hbm.at[fidx_v], data_v, sem_g).wait()

        def out_start(c, data_v):
            for r in range(_CR):
                pltpu.make_async_copy(
                    data_v.at[pl.ds(r * _C, _C)],
                    out_hbm.at[base + c * _CR + r], sem_out).start()

        def out_wait(c, data_v):
            for r in range(_CR):
                pltpu.make_async_copy(
                    data_v.at[pl.ds(r * _C, _C)],
                    out_hbm.at[base + c * _CR + r], sem_out).wait()

        idx_start(0, idx_a)
        idx_start(1, idx_b)
        idx_wait(0, idx_a)
        fidx_compute(idx_a, fidx_a)
        gather_start(fidx_a, data_a)

        def half(c, cur, nxt):
            idx_c, fidx_c, data_c = cur
            idx_n, fidx_n, data_n = nxt

            @pl.when(c + 1 < _NCHUNK)
            def _stage_next():
                idx_wait(c + 1, idx_n)
                fidx_compute(idx_n, fidx_n)

            gather_wait(fidx_c, data_c)

            @pl.when(c > 0)
            def _drain_prev_out():
                out_wait(c - 1, data_n)

            @pl.when(c + 1 < _NCHUNK)
            def _fire_next():
                gather_start(fidx_n, data_n)

            out_start(c, data_c)

            @pl.when(c + 2 < _NCHUNK)
            def _prefetch():
                idx_start(c + 2, idx_c)

        bufs_a = (idx_a, fidx_a, data_a)
        bufs_b = (idx_b, fidx_b, data_b)

        def pair_body(cp, carry):
            half(2 * cp, bufs_a, bufs_b)
            half(2 * cp + 1, bufs_b, bufs_a)
            return carry

        lax.fori_loop(0, _NCHUNK // 2, pair_body, 0)
        out_wait(_NCHUNK - 1, data_b)

    return k(x1d, idx2)


def kernel(x, index):
    x1d = x.reshape(_N)
    out = _sc_gather(x1d, index)
    return out


# speedup vs baseline: 1.4827x; 1.0131x over previous
import functools

import jax
import jax.numpy as jnp
from jax import lax
from jax.experimental import pallas as pl
from jax.experimental.pallas import tpu as pltpu
from jax.experimental.pallas import tpu_sc as plsc

_R, _C = 16384, 4096
_N = _R * _C
_NW = 32
_WROWS = _R // _NW
_CR = 4
_CHUNK = _CR * _C
_HALF = _CHUNK // 2
_NCHUNK = _WROWS // _CR
_SHIFT = 12


def _sc_gather(x1d, idx2):
    mesh = plsc.VectorSubcoreMesh(core_axis_name="c", subcore_axis_name="s")

    @functools.partial(
        pl.kernel,
        mesh=mesh,
        out_type=jax.ShapeDtypeStruct((_R, _C), jnp.float32),
        scratch_types=[
            pltpu.VMEM((_CHUNK,), jnp.int32),
            pltpu.VMEM((_CHUNK,), jnp.int32),
            pltpu.VMEM((_HALF,), jnp.int32),
            pltpu.VMEM((_HALF,), jnp.int32),
            pltpu.VMEM((_HALF,), jnp.int32),
            pltpu.VMEM((_HALF,), jnp.int32),
            pltpu.VMEM((_HALF,), jnp.float32),
            pltpu.VMEM((_HALF,), jnp.float32),
            pltpu.VMEM((_HALF,), jnp.float32),
            pltpu.VMEM((_HALF,), jnp.float32),
            pltpu.SemaphoreType.DMA,
            pltpu.SemaphoreType.DMA,
            pltpu.SemaphoreType.DMA,
            pltpu.SemaphoreType.DMA,
            pltpu.SemaphoreType.DMA,
            pltpu.SemaphoreType.DMA,
        ],
    )
    def k(x_hbm, idx_hbm, out_hbm, idx_a, idx_b,
          fidx_a1, fidx_a2, fidx_b1, fidx_b2,
          data_a1, data_a2, data_b1, data_b2,
          sem_in, sem_out, sem_ga1, sem_ga2, sem_gb1, sem_gb2):
        wid = lax.axis_index("s") * 2 + lax.axis_index("c")
        base = wid * _WROWS
        lane = lax.iota(jnp.int32, 16)

        def idx_start(c, idx_v):
            for r in range(_CR):
                pltpu.make_async_copy(
                    idx_hbm.at[base + c * _CR + r],
                    idx_v.at[pl.ds(r * _C, _C)], sem_in).start()

        def idx_wait(c, idx_v):
            for r in range(_CR):
                pltpu.make_async_copy(
                    idx_hbm.at[base + c * _CR + r],
                    idx_v.at[pl.ds(r * _C, _C)], sem_in).wait()

        def fidx_compute(idx_v, fidx_1, fidx_2):
            def frow(r, carry):
                col = (lax.rem(r, _C // 16) << 4) + lane
                fidx_1[pl.ds(r * 16, 16)] = (
                    (idx_v[pl.ds(r * 16, 16)] << _SHIFT) | col)
                fidx_2[pl.ds(r * 16, 16)] = (
                    (idx_v[pl.ds(_HALF + r * 16, 16)] << _SHIFT) | col)
                return carry
            lax.fori_loop(0, _HALF // 16, frow, 0, unroll=8)

        def gather_start(fidx_v, data_v, sem):
            pltpu.make_async_copy(x_hbm.at[fidx_v], data_v, sem).start()

        def gather_wait(fidx_v, data_v, sem):
            pltpu.make_async_copy(x_hbm.at[fidx_v], data_v, sem).wait()

        def out_start(c, data_1, data_2):
            for r in range(_CR):
                d = data_1 if r < _CR // 2 else data_2
                o = (r % (_CR // 2)) * _C
                pltpu.make_async_copy(
                    d.at[pl.ds(o, _C)],
                    out_hbm.at[base + c * _CR + r], sem_out).start()

        def out_wait(c, data_1, data_2):
            for r in range(_CR):
                d = data_1 if r < _CR // 2 else data_2
                o = (r % (_CR // 2)) * _C
                pltpu.make_async_copy(
                    d.at[pl.ds(o, _C)],
                    out_hbm.at[base + c * _CR + r], sem_out).wait()

        idx_start(0, idx_a)
        idx_start(1, idx_b)
        idx_wait(0, idx_a)
        fidx_compute(idx_a, fidx_a1, fidx_a2)
        gather_start(fidx_a1, data_a1, sem_ga1)
        gather_start(fidx_a2, data_a2, sem_ga2)

        def half(c, cur, nxt):
            (idx_c, fidx_c1, fidx_c2, data_c1, data_c2, sem_c1, sem_c2) = cur
            (idx_n, fidx_n1, fidx_n2, data_n1, data_n2, sem_n1, sem_n2) = nxt

            @pl.when(c + 1 < _NCHUNK)
            def _stage_next():
                idx_wait(c + 1, idx_n)
                fidx_compute(idx_n, fidx_n1, fidx_n2)

            @pl.when(c > 0)
            def _drain_prev_out():
                out_wait(c - 1, data_n1, data_n2)

            @pl.when(c + 1 < _NCHUNK)
            def _fire_next():
                gather_start(fidx_n1, data_n1, sem_n1)
                gather_start(fidx_n2, data_n2, sem_n2)

            gather_wait(fidx_c1, data_c1, sem_c1)
            gather_wait(fidx_c2, data_c2, sem_c2)
            out_start(c, data_c1, data_c2)

            @pl.when(c + 2 < _NCHUNK)
            def _prefetch():
                idx_start(c + 2, idx_c)

        bufs_a = (idx_a, fidx_a1, fidx_a2, data_a1, data_a2, sem_ga1, sem_ga2)
        bufs_b = (idx_b, fidx_b1, fidx_b2, data_b1, data_b2, sem_gb1, sem_gb2)

        def pair_body(cp, carry):
            half(2 * cp, bufs_a, bufs_b)
            half(2 * cp + 1, bufs_b, bufs_a)
            return carry

        lax.fori_loop(0, _NCHUNK // 2, pair_body, 0)
        out_wait(_NCHUNK - 1, data_b1, data_b2)

    return k(x1d, idx2)


def kernel(x, index):
    x1d = x.reshape(_N)
    out = _sc_gather(x1d, index)
    return out
